# baseline (speedup 1.0000x reference)
import functools

import jax
import jax.numpy as jnp
from jax import lax
from jax.experimental import pallas as pl
from jax.experimental.pallas import tpu as pltpu
from jax.experimental.pallas import tpu_sc as plsc

NUM_ENT = 100000
NUM_REL = 500
D = 128
B = 4096
T = 5
QUAL = 8

NC = 2
NS = 16
NW = NC * NS

CHUNK = 128
NBUF = 4

N_HT = B * T * 2
N_Q = B * T * (QUAL // 2)

HT_CHUNKS = N_HT // (NW * CHUNK)
Q_CHUNKS = N_Q // (NW * CHUNK)
Q_PW = Q_CHUNKS * CHUNK

GROUPS = CHUNK // 16

_LANE = None


def _fix_padding(idx_ref, j, rows_ref):
    zeros16 = jnp.zeros((16,), jnp.float32)

    def group_body(g, carry):
        idx16 = idx_ref[pl.ds(j * CHUNK + g * 16, 16)]
        is_zero = idx16 == 0
        nzero = plsc.all_reduce_population_count(is_zero)

        @pl.when(nzero[0] > 0)
        def _():
            lane = lax.iota(jnp.int32, 16)
            for jj in range(16):
                nj = plsc.all_reduce_population_count(is_zero & (lane == jj))

                @pl.when(nj[0] > 0)
                def _():
                    row = g * 16 + jj
                    for c in range(D // 16):
                        rows_ref[row, pl.ds(c * 16, 16)] = zeros16

        return carry

    lax.fori_loop(0, GROUPS, group_body, 0)


def _flatten_idx(src_v, idx_v, tdim, col0, cstride, n):
    per_b = T * tdim

    def dbody(k, carry):
        p = k * 16 + lax.iota(jnp.int32, 16)
        b = p // per_b
        r = p % per_b
        t = r // tdim
        c = col0 + cstride * (r % tdim)
        idx_v[pl.ds(k * 16, 16)] = plsc.load_gather(src_v, [b, t, c])
        return carry

    lax.fori_loop(0, n // 16, dbody, 0)


def _body(ent_hbm, rel_hbm, ht3_hbm, q3_hbm,
          out_ht, out_qr, out_qe, ht_v3, q_v3, idx_v, rows_v, sem_g, sem_w):
    wid = lax.axis_index("s") * NC + lax.axis_index("c")

    bpw = B // NW
    pltpu.sync_copy(ht3_hbm.at[pl.ds(wid * bpw, bpw)], ht_v3)
    pltpu.sync_copy(q3_hbm.at[pl.ds(wid * bpw, bpw)], q_v3)

    for (tab, seg, out_hbm, nch) in (
        (ent_hbm, 0, out_ht, HT_CHUNKS),
        (rel_hbm, 1, out_qr, Q_CHUNKS),
        (ent_hbm, 2, out_qe, Q_CHUNKS),
    ):
        pw = nch * CHUNK
        if seg == 0:
            _flatten_idx(ht_v3, idx_v, 2, 0, 1, pw)
        elif seg == 1:
            _flatten_idx(q_v3, idx_v, QUAL // 2, 0, 2, pw)
        else:
            _flatten_idx(q_v3, idx_v, QUAL // 2, 1, 2, pw)

        def gather(j, b, tab=tab):
            pltpu.async_copy(tab.at[idx_v.at[pl.ds(j * CHUNK, CHUNK)]],
                             rows_v.at[b], sem_g)

        def start_write(j, b, out_hbm=out_hbm, nch=nch):
            base = (wid * nch + j) * CHUNK
            pltpu.async_copy(rows_v.at[b], out_hbm.at[pl.ds(base, CHUNK)],
                             sem_w)

        def wait_write(j, b, out_hbm=out_hbm, nch=nch):
            base = (wid * nch + j) * CHUNK
            pltpu.make_async_copy(rows_v.at[b],
                                  out_hbm.at[pl.ds(base, CHUNK)],
                                  sem_w).wait()

        def wait_gather(j, b, tab=tab):
            pltpu.make_async_copy(tab.at[idx_v.at[pl.ds(j * CHUNK, CHUNK)]],
                                  rows_v.at[b], sem_g).wait()

        for k in range(NBUF - 1):
            gather(k, k)

        def chunk_body(j, carry):
            b = j % NBUF
            wait_gather(j, b)

            @pl.when(j >= 1)
            def _():
                wait_write(j - 1, (j - 1) % NBUF)

            @pl.when(j + NBUF - 1 < nch)
            def _():
                gather(j + NBUF - 1, (j + NBUF - 1) % NBUF)

            _fix_padding(idx_v, j, rows_v.at[b])
            start_write(j, b)
            return carry

        lax.fori_loop(0, nch, chunk_body, 0)
        wait_write(nch - 1, (nch - 1) % NBUF)


@jax.jit
def _run(ht3, q3, ent_embedding, rel_embedding):
    mesh = plsc.VectorSubcoreMesh(core_axis_name="c", subcore_axis_name="s",
                                  num_cores=NC, num_subcores=NS)
    out_type = (
        jax.ShapeDtypeStruct((N_HT, D), jnp.float32),
        jax.ShapeDtypeStruct((N_Q, D), jnp.float32),
        jax.ShapeDtypeStruct((N_Q, D), jnp.float32),
    )
    scratch = [
        pltpu.VMEM((B // NW, T, 2), jnp.int32),
        pltpu.VMEM((B // NW, T, QUAL), jnp.int32),
        pltpu.VMEM((Q_PW,), jnp.int32),
        pltpu.VMEM((NBUF, CHUNK, D), jnp.float32),
        pltpu.SemaphoreType.DMA,
        pltpu.SemaphoreType.DMA,
    ]
    f = pl.kernel(_body, out_type=out_type, mesh=mesh, scratch_types=scratch,
                  compiler_params=pltpu.CompilerParams(
                      needs_layout_passes=False,
                      use_tc_tiling_on_sc=False))
    return f(ent_embedding, rel_embedding, ht3, q3)


def kernel(ht_idx, qual_idx, ent_embedding, rel_embedding):
    out_ht, out_qr, out_qe = _run(ht_idx.astype(jnp.int32),
                                  qual_idx.astype(jnp.int32),
                                  ent_embedding, rel_embedding)
    return (out_ht.reshape(B, T, 2, D),
            out_qr.reshape(B, T, QUAL // 2, D),
            out_qe.reshape(B, T, QUAL // 2, D))


# speedup vs baseline: 1.0462x; 1.0462x over previous
import functools

import jax
import jax.numpy as jnp
from jax import lax
from jax.experimental import pallas as pl
from jax.experimental.pallas import tpu as pltpu
from jax.experimental.pallas import tpu_sc as plsc

NUM_ENT = 100000
NUM_REL = 500
D = 128
B = 4096
T = 5
QUAL = 8

NC = 2
NS = 16
NW = NC * NS

CHUNK = 128
NBUF = 6

N_HT = B * T * 2
N_Q = B * T * (QUAL // 2)

HT_CHUNKS = N_HT // (NW * CHUNK)
Q_CHUNKS = N_Q // (NW * CHUNK)
Q_PW = Q_CHUNKS * CHUNK

GROUPS = CHUNK // 16

_LANE = None


def _fix_padding(idx_ref, j, rows_ref):
    zeros16 = jnp.zeros((16,), jnp.float32)

    def group_body(g, carry):
        idx16 = idx_ref[pl.ds(j * CHUNK + g * 16, 16)]
        is_zero = idx16 == 0
        nzero = plsc.all_reduce_population_count(is_zero)

        @pl.when(nzero[0] > 0)
        def _():
            lane = lax.iota(jnp.int32, 16)
            for jj in range(16):
                nj = plsc.all_reduce_population_count(is_zero & (lane == jj))

                @pl.when(nj[0] > 0)
                def _():
                    row = g * 16 + jj
                    for c in range(D // 16):
                        rows_ref[row, pl.ds(c * 16, 16)] = zeros16

        return carry

    lax.fori_loop(0, GROUPS, group_body, 0)


def _deinterleave(qint_v, idx_v, phase, n):
    def dbody(k, carry):
        lane = lax.iota(jnp.int32, 16)
        src = (k * 16 + lane) * 2 + phase
        idx_v[pl.ds(k * 16, 16)] = plsc.load_gather(qint_v, [src])
        return carry

    lax.fori_loop(0, n // 16, dbody, 0)


def _body(ent_hbm, rel_hbm, comb_hbm,
          out_ht, out_qr, out_qe, qint_v, idx_v, rows_v, sem_g, sem_w):
    wid = lax.axis_index("s") * NC + lax.axis_index("c")

    pltpu.sync_copy(comb_hbm.at[pl.ds(N_HT + wid * 2 * Q_PW, 2 * Q_PW)],
                    qint_v)

    for (tab, seg, out_hbm, nch) in (
        (ent_hbm, 0, out_ht, HT_CHUNKS),
        (rel_hbm, 1, out_qr, Q_CHUNKS),
        (ent_hbm, 2, out_qe, Q_CHUNKS),
    ):
        pw = nch * CHUNK
        if seg == 0:
            pltpu.sync_copy(comb_hbm.at[pl.ds(wid * pw, pw)],
                            idx_v.at[pl.ds(0, pw)])
        else:
            _deinterleave(qint_v, idx_v, seg - 1, pw)

        def gather(j, b, tab=tab):
            pltpu.async_copy(tab.at[idx_v.at[pl.ds(j * CHUNK, CHUNK)]],
                             rows_v.at[b], sem_g)

        def start_write(j, b, out_hbm=out_hbm, nch=nch):
            base = (wid * nch + j) * CHUNK
            pltpu.async_copy(rows_v.at[b], out_hbm.at[pl.ds(base, CHUNK)],
                             sem_w)

        def wait_write(j, b, out_hbm=out_hbm, nch=nch):
            base = (wid * nch + j) * CHUNK
            pltpu.make_async_copy(rows_v.at[b],
                                  out_hbm.at[pl.ds(base, CHUNK)],
                                  sem_w).wait()

        def wait_gather(j, b, tab=tab):
            pltpu.make_async_copy(tab.at[idx_v.at[pl.ds(j * CHUNK, CHUNK)]],
                                  rows_v.at[b], sem_g).wait()

        for k in range(NBUF - 1):
            gather(k, k)

        def chunk_body(j, carry):
            b = j % NBUF
            wait_gather(j, b)

            @pl.when(j >= 1)
            def _():
                wait_write(j - 1, (j - 1) % NBUF)

            @pl.when(j + NBUF - 1 < nch)
            def _():
                gather(j + NBUF - 1, (j + NBUF - 1) % NBUF)

            _fix_padding(idx_v, j, rows_v.at[b])
            start_write(j, b)
            return carry

        lax.fori_loop(0, nch, chunk_body, 0)
        wait_write(nch - 1, (nch - 1) % NBUF)


@jax.jit
def _run(comb_idx, ent_embedding, rel_embedding):
    mesh = plsc.VectorSubcoreMesh(core_axis_name="c", subcore_axis_name="s",
                                  num_cores=NC, num_subcores=NS)
    out_type = (
        jax.ShapeDtypeStruct((N_HT, D), jnp.float32),
        jax.ShapeDtypeStruct((N_Q, D), jnp.float32),
        jax.ShapeDtypeStruct((N_Q, D), jnp.float32),
    )
    scratch = [
        pltpu.VMEM((2 * Q_PW,), jnp.int32),
        pltpu.VMEM((Q_PW,), jnp.int32),
        pltpu.VMEM((NBUF, CHUNK, D), jnp.float32),
        pltpu.SemaphoreType.DMA,
        pltpu.SemaphoreType.DMA,
    ]
    f = pl.kernel(_body, out_type=out_type, mesh=mesh, scratch_types=scratch,
                  compiler_params=pltpu.CompilerParams(
                      needs_layout_passes=False))
    return f(ent_embedding, rel_embedding, comb_idx)


def kernel(ht_idx, qual_idx, ent_embedding, rel_embedding):
    comb_idx = jnp.concatenate(
        [ht_idx.astype(jnp.int32).reshape(N_HT),
         qual_idx.astype(jnp.int32).reshape(2 * N_Q)])
    out_ht, out_qr, out_qe = _run(comb_idx, ent_embedding, rel_embedding)
    return (out_ht.reshape(B, T, 2, D),
            out_qr.reshape(B, T, QUAL // 2, D),
            out_qe.reshape(B, T, QUAL // 2, D))


# speedup vs baseline: 1.0592x; 1.0124x over previous
import functools

import jax
import jax.numpy as jnp
from jax import lax
from jax.experimental import pallas as pl
from jax.experimental.pallas import tpu as pltpu
from jax.experimental.pallas import tpu_sc as plsc

NUM_ENT = 100000
NUM_REL = 500
D = 128
B = 4096
T = 5
QUAL = 8

NC = 2
NS = 16
NW = NC * NS

CHUNK = 128
NBUF = 3
GPB = 2

N_HT = B * T * 2
N_Q = B * T * (QUAL // 2)

HT_CHUNKS = N_HT // (NW * CHUNK)
Q_CHUNKS = N_Q // (NW * CHUNK)
Q_PW = Q_CHUNKS * CHUNK

GROUPS = CHUNK // 16

_LANE = None


def _fix_padding(idx_ref, j, rows_ref):
    zeros16 = jnp.zeros((16,), jnp.float32)

    def group_body(g, carry):
        idx16 = idx_ref[pl.ds(j * CHUNK + g * 16, 16)]
        is_zero = idx16 == 0
        nzero = plsc.all_reduce_population_count(is_zero)

        @pl.when(nzero[0] > 0)
        def _():
            lane = lax.iota(jnp.int32, 16)
            for jj in range(16):
                nj = plsc.all_reduce_population_count(is_zero & (lane == jj))

                @pl.when(nj[0] > 0)
                def _():
                    row = g * 16 + jj
                    for c in range(D // 16):
                        rows_ref[row, pl.ds(c * 16, 16)] = zeros16

        return carry

    lax.fori_loop(0, GROUPS, group_body, 0)


def _deinterleave(qint_v, idx_v, phase, n):
    def dbody(k, carry):
        lane = lax.iota(jnp.int32, 16)
        src = (k * 16 + lane) * 2 + phase
        idx_v[pl.ds(k * 16, 16)] = plsc.load_gather(qint_v, [src])
        return carry

    lax.fori_loop(0, n // 16, dbody, 0)


def _body(ent_hbm, rel_hbm, comb_hbm,
          out_ht, out_qr, out_qe, qint_v, idx_v, rows_v, sem_g, sem_w):
    wid = lax.axis_index("s") * NC + lax.axis_index("c")

    pltpu.sync_copy(comb_hbm.at[pl.ds(N_HT + wid * 2 * Q_PW, 2 * Q_PW)],
                    qint_v)

    for (tab, seg, out_hbm, nch) in (
        (ent_hbm, 0, out_ht, HT_CHUNKS),
        (rel_hbm, 1, out_qr, Q_CHUNKS),
        (ent_hbm, 2, out_qe, Q_CHUNKS),
    ):
        pw = nch * CHUNK
        if seg == 0:
            pltpu.sync_copy(comb_hbm.at[pl.ds(wid * pw, pw)],
                            idx_v.at[pl.ds(0, pw)])
        else:
            _deinterleave(qint_v, idx_v, seg - 1, pw)

        nsup = nch // GPB

        def gather(s, b, tab=tab):
            for h in range(GPB):
                j = s * GPB + h
                pltpu.async_copy(tab.at[idx_v.at[pl.ds(j * CHUNK, CHUNK)]],
                                 rows_v.at[b].at[pl.ds(h * CHUNK, CHUNK)],
                                 sem_g)

        def wait_gather(s, b, tab=tab):
            for h in range(GPB):
                j = s * GPB + h
                pltpu.make_async_copy(
                    tab.at[idx_v.at[pl.ds(j * CHUNK, CHUNK)]],
                    rows_v.at[b].at[pl.ds(h * CHUNK, CHUNK)],
                    sem_g).wait()

        def start_write(s, b, out_hbm=out_hbm, nch=nch):
            base = (wid * nch + s * GPB) * CHUNK
            pltpu.async_copy(rows_v.at[b],
                             out_hbm.at[pl.ds(base, GPB * CHUNK)], sem_w)

        def wait_write(s, b, out_hbm=out_hbm, nch=nch):
            base = (wid * nch + s * GPB) * CHUNK
            pltpu.make_async_copy(rows_v.at[b],
                                  out_hbm.at[pl.ds(base, GPB * CHUNK)],
                                  sem_w).wait()

        for k in range(NBUF - 1):
            gather(k, k)

        def chunk_body(s, carry):
            b = s % NBUF
            wait_gather(s, b)

            @pl.when(s >= 1)
            def _():
                wait_write(s - 1, (s - 1) % NBUF)

            @pl.when(s + NBUF - 1 < nsup)
            def _():
                gather(s + NBUF - 1, (s + NBUF - 1) % NBUF)

            for h in range(GPB):
                _fix_padding(idx_v, s * GPB + h,
                             rows_v.at[b].at[pl.ds(h * CHUNK, CHUNK)])
            start_write(s, b)
            return carry

        lax.fori_loop(0, nsup, chunk_body, 0)
        wait_write(nsup - 1, (nsup - 1) % NBUF)


@jax.jit
def _run(comb_idx, ent_embedding, rel_embedding):
    mesh = plsc.VectorSubcoreMesh(core_axis_name="c", subcore_axis_name="s",
                                  num_cores=NC, num_subcores=NS)
    out_type = (
        jax.ShapeDtypeStruct((N_HT, D), jnp.float32),
        jax.ShapeDtypeStruct((N_Q, D), jnp.float32),
        jax.ShapeDtypeStruct((N_Q, D), jnp.float32),
    )
    scratch = [
        pltpu.VMEM((2 * Q_PW,), jnp.int32),
        pltpu.VMEM((Q_PW,), jnp.int32),
        pltpu.VMEM((NBUF, GPB * CHUNK, D), jnp.float32),
        pltpu.SemaphoreType.DMA,
        pltpu.SemaphoreType.DMA,
    ]
    f = pl.kernel(_body, out_type=out_type, mesh=mesh, scratch_types=scratch,
                  compiler_params=pltpu.CompilerParams(
                      needs_layout_passes=False))
    return f(ent_embedding, rel_embedding, comb_idx)


def kernel(ht_idx, qual_idx, ent_embedding, rel_embedding):
    comb_idx = jnp.concatenate(
        [ht_idx.astype(jnp.int32).reshape(N_HT),
         qual_idx.astype(jnp.int32).reshape(2 * N_Q)])
    out_ht, out_qr, out_qe = _run(comb_idx, ent_embedding, rel_embedding)
    return (out_ht.reshape(B, T, 2, D),
            out_qr.reshape(B, T, QUAL // 2, D),
            out_qe.reshape(B, T, QUAL // 2, D))
